# baseline (speedup 1.0000x reference)
import jax.numpy as jnp

import kernel_sc as _sc

_N, _C, _M = 1024, 81, 28
_P = _M * _M


def kernel(mask_logits, labels, mask_targets):
    xt = jnp.transpose(mask_logits, (2, 3, 1, 0))
    tt = jnp.transpose(mask_targets, (1, 2, 0))
    parts = _sc.sc_partial_sums(xt, labels.astype(jnp.int32), tt)
    return jnp.sum(parts) / (_N * _P)


# speedup vs baseline: 1.4013x; 1.4013x over previous
import jax
import jax.numpy as jnp
from jax import lax
from jax.experimental import pallas as pl
from jax.experimental.pallas import tpu as pltpu
from jax.experimental.pallas import tpu_sc as plsc

_N, _C, _M = 1024, 81, 28
_P = _M * _M
_MT = 18
_P0 = _MT * _M
_PSC = _P - _P0
_NW = 32
_NG = _N // 16

_L1P = (
    0.9999987635044436,
    -0.499871915934771,
    0.33112051909778917,
    -0.23514863754146653,
    0.14943458362588757,
    -0.06658804993609893,
    0.014202825621286636,
)


def _log1p_poly(u):
    q = jnp.full((16,), _L1P[6], jnp.float32)
    for c in _L1P[5::-1]:
        q = q * u + c
    return u * q


def _sc_body(x_hbm, lbl_hbm, t_hbm, out_hbm, slab, tslab, lblv, accv):
    wid = lax.axis_index("s") * 2 + lax.axis_index("c")
    pltpu.sync_copy(lbl_hbm, lblv)
    n_k = jnp.where(wid < _PSC - 8 * _NW, 9, 8)

    def pos_body(k, acc):
        p = _P0 + k * _NW + wid
        m1 = p // _M
        m2 = p % _M
        pltpu.sync_copy(x_hbm.at[m1, m2], slab)
        pltpu.sync_copy(t_hbm.at[m1, pl.ds(m2, 1)], tslab)

        def grp_body(g, acc_g):
            off = g * 16
            nidx = lax.iota(jnp.int32, 16) + off
            lbl16 = lblv[pl.ds(off, 16)]
            x16 = plsc.load_gather(slab, [lbl16, nidx])
            t16 = tslab[0, pl.ds(off, 16)]
            u = jnp.exp(-jnp.abs(x16))
            bce = jnp.maximum(x16, 0.0) - x16 * t16 + _log1p_poly(u)
            return acc_g + bce

        return lax.fori_loop(0, _NG, grp_body, acc)

    acc = lax.fori_loop(0, n_k, pos_body, jnp.zeros((16,), jnp.float32))
    accv[...] = acc
    pltpu.sync_copy(accv, out_hbm.at[pl.ds(wid * 16, 16)])


def _tc_body(labels_ref, x_ref, t_ref, out_ref):
    i = pl.program_id(0)
    lbl = labels_ref[...]
    ci = lax.broadcasted_iota(jnp.int32, (_C, _N), 0)
    oh = (ci == lbl).astype(jnp.float32)
    x = x_ref[0]
    sel = jnp.sum(x * oh[None], axis=1)
    t = t_ref[0]
    bce = jnp.maximum(sel, 0.0) - sel * t + jnp.log1p(jnp.exp(-jnp.abs(sel)))
    s = jnp.sum(bce)
    prev = jnp.where(i == 0, 0.0, out_ref[0, 0])
    out_ref[0, 0] = prev + s


def kernel(mask_logits, labels, mask_targets):
    xt = jnp.transpose(mask_logits, (2, 3, 1, 0))
    tt = jnp.transpose(mask_targets, (1, 2, 0))
    lbl = labels.astype(jnp.int32)

    sc_fn = pl.kernel(
        _sc_body,
        out_type=jax.ShapeDtypeStruct((_NW * 16,), jnp.float32),
        mesh=plsc.VectorSubcoreMesh(core_axis_name="c", subcore_axis_name="s"),
        scratch_types=[
            pltpu.VMEM((_C, _N), jnp.float32),
            pltpu.VMEM((1, _N), jnp.float32),
            pltpu.VMEM((_N,), jnp.int32),
            pltpu.VMEM((16,), jnp.float32),
        ],
        compiler_params=pltpu.CompilerParams(needs_layout_passes=False),
    )
    sc_parts = sc_fn(xt, lbl, tt)

    grid_spec = pltpu.PrefetchScalarGridSpec(
        num_scalar_prefetch=0,
        grid=(_MT,),
        in_specs=[
            pl.BlockSpec((1, _N), lambda i: (0, 0)),
            pl.BlockSpec((1, _M, _C, _N), lambda i: (i, 0, 0, 0)),
            pl.BlockSpec((1, _M, _N), lambda i: (i, 0, 0)),
        ],
        out_specs=pl.BlockSpec(memory_space=pltpu.SMEM),
    )
    tc_sum = pl.pallas_call(
        _tc_body,
        grid_spec=grid_spec,
        out_shape=jax.ShapeDtypeStruct((1, 1), jnp.float32),
    )(lbl.reshape(1, _N), xt, tt)

    return (tc_sum[0, 0] + jnp.sum(sc_parts)) / (_N * _P)
